# Initial kernel scaffold; baseline (speedup 1.0000x reference)
import jax
import jax.numpy as jnp
from jax.experimental import pallas as pl


def kernel(x, embedding):
    raise NotImplementedError("write your pallas kernel here")


# speedup vs baseline: 1.5479x; 1.5479x over previous
import functools

import jax
import jax.numpy as jnp
from jax import lax
from jax.experimental import pallas as pl
from jax.experimental.pallas import tpu as pltpu
from jax.experimental.pallas import tpu_sc as plsc

DIM = 32
NUM_WORKERS = 32
CHUNK = 1024


def _gather_kernel(n_chunks, b_per_w, table_hbm, idx_hbm, out_hbm,
                   idx_v, rows_v, sem):
    wid = lax.axis_index("s") * 2 + lax.axis_index("c")
    base = wid * b_per_w

    def body(j, carry):
        off = pl.multiple_of(base + j * CHUNK, CHUNK)
        pltpu.sync_copy(idx_hbm.at[pl.ds(off, CHUNK)], idx_v)
        pltpu.async_copy(table_hbm.at[idx_v], rows_v, sem).wait()
        pltpu.sync_copy(rows_v, out_hbm.at[pl.ds(off, CHUNK)])
        return carry

    lax.fori_loop(0, n_chunks, body, 0)


def kernel(x, embedding):
    batch, fields = x.shape
    b = batch * fields
    assert b % (NUM_WORKERS * CHUNK) == 0
    b_per_w = b // NUM_WORKERS
    n_chunks = b_per_w // CHUNK

    idx = x.reshape(b)
    mesh = plsc.VectorSubcoreMesh(core_axis_name="c", subcore_axis_name="s")

    run = pl.kernel(
        functools.partial(_gather_kernel, n_chunks, b_per_w),
        out_type=jax.ShapeDtypeStruct((b, DIM), jnp.float32),
        mesh=mesh,
        scratch_types=[
            pltpu.VMEM((CHUNK,), jnp.int32),
            pltpu.VMEM((CHUNK, DIM), jnp.float32),
            pltpu.SemaphoreType.DMA,
        ],
        compiler_params=pltpu.CompilerParams(use_tc_tiling_on_sc=False),
    )
    out = run(embedding, idx)
    return out.reshape(batch, fields, DIM)


# speedup vs baseline: 1.5754x; 1.0178x over previous
import functools

import jax
import jax.numpy as jnp
from jax import lax
from jax.experimental import pallas as pl
from jax.experimental.pallas import tpu as pltpu
from jax.experimental.pallas import tpu_sc as plsc

DIM = 32
NUM_WORKERS = 32
CHUNK = 1664
N_CHUNKS = 8


def _gather_kernel(b_per_w, table_hbm, idx_hbm, out_hbm,
                   idx_v, rows0, rows1, gsem0, gsem1, wsem0, wsem1):
    wid = lax.axis_index("s") * 2 + lax.axis_index("c")
    base = wid * b_per_w
    pltpu.sync_copy(idx_hbm.at[pl.ds(base, b_per_w)], idx_v)

    rows = (rows0, rows1)
    gsem = (gsem0, gsem1)
    wsem = (wsem0, wsem1)
    gh = [None, None]
    wh = [None, None]

    def writeback(c):
        b = c & 1
        gh[b].wait()
        wh[b] = pltpu.async_copy(
            rows[b], out_hbm.at[pl.ds(base + c * CHUNK, CHUNK)], wsem[b])

    for c in range(N_CHUNKS):
        b = c & 1
        if wh[b] is not None:
            wh[b].wait()
        gh[b] = pltpu.async_copy(
            table_hbm.at[idx_v.at[pl.ds(c * CHUNK, CHUNK)]], rows[b], gsem[b])
        if c >= 1:
            writeback(c - 1)
    writeback(N_CHUNKS - 1)
    wh[0].wait()
    wh[1].wait()


def kernel(x, embedding):
    batch, fields = x.shape
    b = batch * fields
    assert b == NUM_WORKERS * N_CHUNKS * CHUNK
    b_per_w = b // NUM_WORKERS

    idx = x.reshape(b)
    mesh = plsc.VectorSubcoreMesh(core_axis_name="c", subcore_axis_name="s")

    run = pl.kernel(
        functools.partial(_gather_kernel, b_per_w),
        out_type=jax.ShapeDtypeStruct((b, DIM), jnp.float32),
        mesh=mesh,
        scratch_types=[
            pltpu.VMEM((b_per_w,), jnp.int32),
            pltpu.VMEM((CHUNK, DIM), jnp.float32),
            pltpu.VMEM((CHUNK, DIM), jnp.float32),
            pltpu.SemaphoreType.DMA,
            pltpu.SemaphoreType.DMA,
            pltpu.SemaphoreType.DMA,
            pltpu.SemaphoreType.DMA,
        ],
        compiler_params=pltpu.CompilerParams(use_tc_tiling_on_sc=False),
    )
    out = run(embedding, idx)
    return out.reshape(batch, fields, DIM)
